# Initial kernel scaffold; baseline (speedup 1.0000x reference)
import jax
import jax.numpy as jnp
from jax.experimental import pallas as pl


def kernel(x, W, b):
    raise NotImplementedError("write your pallas kernel here")


# speedup vs baseline: 2.7662x; 2.7662x over previous
import functools

import jax
import jax.numpy as jnp
from jax import lax
from jax.experimental import pallas as pl
from jax.experimental.pallas import tpu as pltpu
from jax.experimental.pallas import tpu_sc as plsc

_NC = 2
_NS = 16
_L = 16
_NW = _NC * _NS


def _tc_logits_body(x_ref, wt_ref, b_ref, out_ref):
    p = jnp.dot(x_ref[...], wt_ref[...], preferred_element_type=jnp.float32)
    out_ref[0] = p.T + b_ref[...]


def _sc_top1_body(lt_hbm, score_hbm, idx_hbm, lbuf, sbuf, ibuf):
    E = lbuf.shape[0]
    tpw = lbuf.shape[1]
    wid = lax.axis_index("s") * _NC + lax.axis_index("c")
    pltpu.sync_copy(lt_hbm.at[wid], lbuf)

    def step(j, _):
        off = j * _L
        ls = [lbuf[e, pl.ds(off, _L)] for e in range(E)]
        m = ls[0]
        idx = jnp.zeros((_L,), jnp.int32)
        for e in range(1, E):
            g = ls[e] > m
            m = jnp.where(g, ls[e], m)
            idx = jnp.where(g, jnp.full((_L,), e, jnp.int32), idx)
        s = jnp.exp(ls[0] - m)
        for e in range(1, E):
            s = s + jnp.exp(ls[e] - m)
        sbuf[pl.ds(off, _L)] = 1.0 / s
        ibuf[pl.ds(off, _L)] = idx
        return 0

    lax.fori_loop(0, tpw // _L, step, 0)
    base = wid * tpw
    pltpu.sync_copy(sbuf, score_hbm.at[pl.ds(base, tpw)])
    pltpu.sync_copy(ibuf, idx_hbm.at[pl.ds(base, tpw)])


def kernel(x, W, b):
    d_model = x.shape[-1]
    n_experts = W.shape[0]
    x_flat = x.reshape(-1, d_model)
    n_tok = x_flat.shape[0]
    tpw = n_tok // _NW

    logits_t = pl.pallas_call(
        _tc_logits_body,
        grid=(_NW,),
        in_specs=[
            pl.BlockSpec((tpw, d_model), lambda i: (i, 0)),
            pl.BlockSpec((d_model, n_experts), lambda i: (0, 0)),
            pl.BlockSpec((n_experts, 1), lambda i: (0, 0)),
        ],
        out_specs=pl.BlockSpec((1, n_experts, tpw), lambda i: (i, 0, 0)),
        out_shape=jax.ShapeDtypeStruct((_NW, n_experts, tpw), jnp.float32),
    )(x_flat, W.T, b.reshape(n_experts, 1))

    mesh = plsc.VectorSubcoreMesh(core_axis_name="c", subcore_axis_name="s")
    scores, idx = pl.kernel(
        _sc_top1_body,
        out_type=(
            jax.ShapeDtypeStruct((n_tok,), jnp.float32),
            jax.ShapeDtypeStruct((n_tok,), jnp.int32),
        ),
        mesh=mesh,
        scratch_types=[
            pltpu.VMEM((n_experts, tpw), jnp.float32),
            pltpu.VMEM((tpw,), jnp.float32),
            pltpu.VMEM((tpw,), jnp.int32),
        ],
    )(logits_t)

    return scores.reshape(n_tok, 1), idx.reshape(n_tok, 1)


# speedup vs baseline: 3.2581x; 1.1778x over previous
import functools

import jax
import jax.numpy as jnp
from jax import lax
from jax.experimental import pallas as pl
from jax.experimental.pallas import tpu as pltpu
from jax.experimental.pallas import tpu_sc as plsc

_NC = 2
_NS = 16
_L = 16
_NW = _NC * _NS


def _tc_logits_body(x_ref, wt_ref, b_ref, out_ref):
    p = jnp.dot(x_ref[...], wt_ref[...], preferred_element_type=jnp.float32)
    out_ref[...] = p.T + b_ref[...]


def _sc_top1_body(lt_hbm, score_hbm, idx_hbm, lbuf, sbuf, ibuf):
    E = lbuf.shape[0]
    tpw = lbuf.shape[1]
    wid = lax.axis_index("s") * _NC + lax.axis_index("c")
    pltpu.sync_copy(lt_hbm.at[:, pl.ds(wid * tpw, tpw)], lbuf)

    def step(j, _):
        off = j * _L
        ls = [lbuf[e, pl.ds(off, _L)] for e in range(E)]
        m = ls[0]
        idx = jnp.zeros((_L,), jnp.int32)
        for e in range(1, E):
            g = ls[e] > m
            m = jnp.where(g, ls[e], m)
            idx = jnp.where(g, jnp.full((_L,), e, jnp.int32), idx)
        s = jnp.exp(ls[0] - m)
        for e in range(1, E):
            s = s + jnp.exp(ls[e] - m)
        sbuf[pl.ds(off, _L)] = 1.0 / s
        ibuf[pl.ds(off, _L)] = idx
        return 0

    lax.fori_loop(0, tpw // _L, step, 0)
    base = wid * tpw
    pltpu.sync_copy(sbuf, score_hbm.at[pl.ds(base, tpw)])
    pltpu.sync_copy(ibuf, idx_hbm.at[pl.ds(base, tpw)])


def kernel(x, W, b):
    d_model = x.shape[-1]
    n_experts = W.shape[0]
    x_flat = x.reshape(-1, d_model)
    n_tok = x_flat.shape[0]
    tpw = n_tok // _NW
    bm = 4096

    logits_t = pl.pallas_call(
        _tc_logits_body,
        grid=(n_tok // bm,),
        in_specs=[
            pl.BlockSpec((bm, d_model), lambda i: (i, 0)),
            pl.BlockSpec((d_model, n_experts), lambda i: (0, 0)),
            pl.BlockSpec((n_experts, 1), lambda i: (0, 0)),
        ],
        out_specs=pl.BlockSpec((n_experts, bm), lambda i: (0, i)),
        out_shape=jax.ShapeDtypeStruct((n_experts, n_tok), jnp.float32),
    )(x_flat, W.T, b.reshape(n_experts, 1))

    mesh = plsc.VectorSubcoreMesh(core_axis_name="c", subcore_axis_name="s")
    scores, idx = pl.kernel(
        _sc_top1_body,
        out_type=(
            jax.ShapeDtypeStruct((n_tok,), jnp.float32),
            jax.ShapeDtypeStruct((n_tok,), jnp.int32),
        ),
        mesh=mesh,
        scratch_types=[
            pltpu.VMEM((n_experts, tpw), jnp.float32),
            pltpu.VMEM((tpw,), jnp.float32),
            pltpu.VMEM((tpw,), jnp.int32),
        ],
    )(logits_t)

    return scores.reshape(n_tok, 1), idx.reshape(n_tok, 1)
